# baseline (speedup 1.0000x reference)
import functools

import jax
import jax.numpy as jnp
from jax import lax
from jax.experimental import pallas as pl
from jax.experimental.pallas import tpu as pltpu

_BLK = 512
_NCH = 1
_CH = _BLK // _NCH


def _mm(a, b):
    return jnp.dot(a.astype(jnp.bfloat16), b,
                   preferred_element_type=jnp.float32)


def _ln(x, g, b):
    m = jnp.mean(x, axis=-1, keepdims=True)
    v = jnp.var(x, axis=-1, keepdims=True)
    return (x - m) / jnp.sqrt(v + 1e-5) * g + b


def _split3(cb):
    def trunc(a):
        u = lax.bitcast_convert_type(a, jnp.uint32)
        return lax.bitcast_convert_type(u & jnp.uint32(0xFFFF0000), jnp.float32)
    c1 = trunc(cb)
    r1 = cb - c1
    c2 = trunc(r1)
    r2 = r1 - c2
    return c1.astype(jnp.bfloat16), c2.astype(jnp.bfloat16), r2.astype(jnp.bfloat16)


def _body(n_total, x_ref, ew1, eb1, eg1, ebe1, ew2, eb2, cbt_ref,
          cbtb_ref, gw_ref,
          dw1, db1, dg1, dbe1, dw2, db2,
          rec_ref, idx_ref, commit_ref, hist_ref, perp_ref, qv_ref):
    i = pl.program_id(0)
    nsteps = pl.num_programs(0)

    @pl.when(i == 0)
    def _init():
        commit_ref[...] = jnp.zeros_like(commit_ref)
        hist_ref[...] = jnp.zeros_like(hist_ref)
        perp_ref[...] = jnp.zeros_like(perp_ref)

    nq, l_dim, k_dim = cbt_ref.shape
    lane_iota = lax.broadcasted_iota(jnp.int32, (_CH, k_dim), 1)
    dn = (((1,), (1,)), ((), ()))

    zs, res, tq = [], [], []
    for c in range(_NCH):
        x = x_ref[pl.ds(c * _CH, _CH), :]
        h = _ln(_mm(x, ew1[...]) + eb1[...], eg1[...], ebe1[...])
        z = _mm(jax.nn.gelu(h), ew2[...]) + eb2[...]
        zs.append(z)
        res.append(z)
        tq.append(jnp.zeros_like(z))

    commit_sum = jnp.float32(0.0)
    for q in range(nq):
        cbt = cbt_ref[q]
        n_row = jnp.sum(cbt * cbt, axis=0, keepdims=True)
        cbtb = cbtb_ref[q]
        for c in range(_NCH):
            r = res[c]
            rsq = jnp.sum(r * r, axis=-1, keepdims=True)
            mm2 = jnp.dot((r + r).astype(jnp.bfloat16), cbtb,
                          preferred_element_type=jnp.float32)
            d = (rsq - mm2) + n_row
            dmin = jnp.min(d, axis=-1, keepdims=True)
            eq32 = (d == dmin).astype(jnp.float32)
            out = lax.dot_general(eq32.astype(jnp.bfloat16), gw_ref[q], dn,
                                  preferred_element_type=jnp.float32)
            g3 = 3 * l_dim
            sbits = lax.bitcast_convert_type(out[:, g3:g3 + 4], jnp.int32)
            e = 254 - (sbits >> 23)
            goff = lax.broadcasted_iota(jnp.int32, e.shape, 1) * 128
            idx = jnp.min(jnp.where(e < 128, e + goff, k_dim), axis=-1)
            cnt = out[:, g3 + 4]
            qv_ref[...] = (out[:, :l_dim] + out[:, l_dim:2 * l_dim]
                           + out[:, 2 * l_dim:g3])
            if q == 0:
                hist_ref[...] += jnp.sum(eq32, axis=0, keepdims=True)

            @pl.when(jnp.any(cnt > 1.5))
            def _fix_ties():
                onehot32 = (lane_iota == idx[:, None]).astype(jnp.float32)
                parts = lax.dot_general(onehot32.astype(jnp.bfloat16),
                                        gw_ref[q], dn,
                                        preferred_element_type=jnp.float32)
                qv_ref[...] = (parts[:, :l_dim] + parts[:, l_dim:2 * l_dim]
                               + parts[:, 2 * l_dim:g3])
                if q == 0:
                    hist_ref[...] += jnp.sum(onehot32 - eq32, axis=0,
                                             keepdims=True)

            qv = qv_ref[...]
            commit_sum += jnp.sum((qv - r) ** 2)
            tq[c] = tq[c] + qv
            res[c] = r - qv
            idx_ref[pl.ds(c * _CH, _CH), q] = idx

    for c in range(_NCH):
        quantized = zs[c] + (tq[c] - zs[c])
        hd = _ln(_mm(quantized, dw1[...]) + db1[...], dg1[...], dbe1[...])
        rec_ref[pl.ds(c * _CH, _CH), :] = _mm(jax.nn.gelu(hd), dw2[...]) + db2[...]

    commit_ref[...] += jnp.full((1, 1), 0.25 / (n_total * l_dim)) * commit_sum

    @pl.when(i == nsteps - 1)
    def _finish():
        p = hist_ref[...] / jnp.float32(n_total)
        perp_ref[...] = jnp.exp(-jnp.sum(p * jnp.log(p + 1e-10),
                                         keepdims=True))


def kernel(x, enc_W1, enc_b1, enc_g1, enc_be1, enc_W2, enc_b2, codebooks,
           dec_W1, dec_b1, dec_g1, dec_be1, dec_W2, dec_b2):
    B, T, D = x.shape
    Q, K, L = codebooks.shape
    H = enc_W1.shape[1]
    N = B * T
    xf = x.reshape(N, D)
    cbt = jnp.swapaxes(codebooks, 1, 2)
    cbt1, cbt2, cbt3 = _split3(cbt)
    cbt123 = jnp.concatenate([cbt1, cbt2, cbt3], axis=1)
    jj = jnp.arange(K)[None, :]
    gg = jnp.arange(K // 128)[:, None]
    wexp = jnp.where(jj // 128 == gg,
                     jnp.exp2((127 - (jj - 128 * gg)).astype(jnp.float32)),
                     0.0).astype(jnp.bfloat16)
    gw = jnp.concatenate(
        [cbt123,
         jnp.broadcast_to(wexp[None], (Q,) + wexp.shape),
         jnp.ones((Q, 1, K), jnp.bfloat16)], axis=1)
    bf = lambda a: a.astype(jnp.bfloat16)
    row = lambda a: a.reshape(1, -1)
    grid = N // _BLK

    full = lambda shape: pl.BlockSpec(shape, lambda i: (0,) * len(shape))
    out_shapes = (
        jax.ShapeDtypeStruct((N, D), jnp.float32),
        jax.ShapeDtypeStruct((N, Q), jnp.int32),
        jax.ShapeDtypeStruct((1, 1), jnp.float32),
        jax.ShapeDtypeStruct((1, K), jnp.float32),
        jax.ShapeDtypeStruct((1, 1), jnp.float32),
    )
    rec, idx, commit, _hist, perp = pl.pallas_call(
        functools.partial(_body, N),
        grid=(grid,),
        in_specs=[
            pl.BlockSpec((_BLK, D), lambda i: (i, 0)),
            full((D, H)), full((1, H)), full((1, H)), full((1, H)),
            full((H, L)), full((1, L)),
            full((Q, L, K)), full((Q, L, K)), full((Q, 3 * L + K // 128 + 1, K)),
            full((L, H)), full((1, H)), full((1, H)), full((1, H)),
            full((H, D)), full((1, D)),
        ],
        out_specs=(
            pl.BlockSpec((_BLK, D), lambda i: (i, 0)),
            pl.BlockSpec((_BLK, Q), lambda i: (i, 0)),
            full((1, 1)), full((1, K)), full((1, 1)),
        ),
        out_shape=out_shapes,
        scratch_shapes=[pltpu.VMEM((_CH, L), jnp.float32)],
    )(xf, bf(enc_W1), row(enc_b1), row(enc_g1), row(enc_be1), bf(enc_W2),
      row(enc_b2), cbt, bf(cbt), gw,
      bf(dec_W1), row(dec_b1), row(dec_g1), row(dec_be1),
      bf(dec_W2), row(dec_b2))
    return (rec.reshape(B, T, D), idx.reshape(B, T, Q),
            commit[0, 0], perp[0, 0])


# speedup vs baseline: 1.6456x; 1.6456x over previous
import functools

import jax
import jax.numpy as jnp
from jax import lax
from jax.experimental import pallas as pl
from jax.experimental.pallas import tpu as pltpu

_BLK = 512
_NCH = 1
_CH = _BLK // _NCH


def _mm(a, b):
    return jnp.dot(a.astype(jnp.bfloat16), b,
                   preferred_element_type=jnp.float32)


def _ln(x, g, b):
    m = jnp.mean(x, axis=-1, keepdims=True)
    v = jnp.var(x, axis=-1, keepdims=True)
    return (x - m) / jnp.sqrt(v + 1e-5) * g + b


def _split3(cb):
    def trunc(a):
        u = lax.bitcast_convert_type(a, jnp.uint32)
        return lax.bitcast_convert_type(u & jnp.uint32(0xFFFF0000), jnp.float32)
    c1 = trunc(cb)
    r1 = cb - c1
    c2 = trunc(r1)
    r2 = r1 - c2
    return c1.astype(jnp.bfloat16), c2.astype(jnp.bfloat16), r2.astype(jnp.bfloat16)


def _body(n_total, x_ref, ew1, eb1, eg1, ebe1, ew2, eb2, cbt_ref,
          cbtb_ref, cbt123_ref,
          dw1, db1, dg1, dbe1, dw2, db2,
          rec_ref, idx_ref, commit_ref, hist_ref, perp_ref, nrow_ref):
    i = pl.program_id(0)
    nsteps = pl.num_programs(0)

    @pl.when(i == 0)
    def _init():
        commit_ref[...] = jnp.zeros_like(commit_ref)
        hist_ref[...] = jnp.zeros_like(hist_ref)
        perp_ref[...] = jnp.zeros_like(perp_ref)
        cba = cbt_ref[...]
        nrow_ref[...] = jnp.sum(cba * cba, axis=1)

    nq, l_dim, k_dim = cbt_ref.shape
    lane_iota = lax.broadcasted_iota(jnp.int32, (_CH, k_dim), 1)
    dn = (((1,), (1,)), ((), ()))

    zs, res, tq = [], [], []
    for c in range(_NCH):
        x = x_ref[pl.ds(c * _CH, _CH), :]
        h = _ln(_mm(x, ew1[...]) + eb1[...], eg1[...], ebe1[...])
        z = _mm(jax.nn.gelu(h), ew2[...]) + eb2[...]
        zs.append(z)
        res.append(z)
        tq.append(jnp.zeros_like(z))

    sq_acc = [jnp.zeros((_CH, l_dim), jnp.float32) for _ in range(_NCH)]
    for q in range(nq):
        n_row = nrow_ref[pl.ds(q, 1), :]
        cbtb = cbtb_ref[q]
        parts_w = cbt123_ref[q]
        for c in range(_NCH):
            r = res[c]
            rsq = jnp.sum(r * r, axis=-1, keepdims=True)
            mm2 = jnp.dot((r + r).astype(jnp.bfloat16), cbtb,
                          preferred_element_type=jnp.float32)
            d = (rsq - mm2) + n_row
            dmin = jnp.min(d, axis=-1, keepdims=True)
            idx = jnp.min(jnp.where(d == dmin, lane_iota, k_dim), axis=-1)
            onehot32 = (lane_iota == idx[:, None]).astype(jnp.float32)
            onehot = onehot32.astype(jnp.bfloat16)
            parts = lax.dot_general(onehot, parts_w, dn,
                                    preferred_element_type=jnp.float32)
            qv = (parts[:, :l_dim] + parts[:, l_dim:2 * l_dim]
                  + parts[:, 2 * l_dim:])
            sq_acc[c] = sq_acc[c] + (qv - r) ** 2
            tq[c] = tq[c] + qv
            res[c] = r - qv
            idx_ref[pl.ds(c * _CH, _CH), q] = idx
            if q == 0:
                hist_ref[...] += jnp.sum(onehot32, axis=0, keepdims=True)

    for c in range(_NCH):
        quantized = zs[c] + (tq[c] - zs[c])
        hd = _ln(_mm(quantized, dw1[...]) + db1[...], dg1[...], dbe1[...])
        rec_ref[pl.ds(c * _CH, _CH), :] = _mm(jax.nn.gelu(hd), dw2[...]) + db2[...]

    commit_sum = sum(jnp.sum(a) for a in sq_acc)
    commit_ref[...] += jnp.full((1, 1), 0.25 / (n_total * l_dim)) * commit_sum

    @pl.when(i == nsteps - 1)
    def _finish():
        p = hist_ref[...] / jnp.float32(n_total)
        perp_ref[...] = jnp.exp(-jnp.sum(p * jnp.log(p + 1e-10),
                                         keepdims=True))


def kernel(x, enc_W1, enc_b1, enc_g1, enc_be1, enc_W2, enc_b2, codebooks,
           dec_W1, dec_b1, dec_g1, dec_be1, dec_W2, dec_b2):
    B, T, D = x.shape
    Q, K, L = codebooks.shape
    H = enc_W1.shape[1]
    N = B * T
    xf = x.reshape(N, D)
    cbt = jnp.swapaxes(codebooks, 1, 2)
    cbt1, cbt2, cbt3 = _split3(cbt)
    cbt123 = jnp.concatenate([cbt1, cbt2, cbt3], axis=1)
    bf = lambda a: a.astype(jnp.bfloat16)
    row = lambda a: a.reshape(1, -1)
    grid = N // _BLK

    full = lambda shape: pl.BlockSpec(shape, lambda i: (0,) * len(shape))
    out_shapes = (
        jax.ShapeDtypeStruct((N, D), jnp.float32),
        jax.ShapeDtypeStruct((N, Q), jnp.int32),
        jax.ShapeDtypeStruct((1, 1), jnp.float32),
        jax.ShapeDtypeStruct((1, K), jnp.float32),
        jax.ShapeDtypeStruct((1, 1), jnp.float32),
    )
    rec, idx, commit, _hist, perp = pl.pallas_call(
        functools.partial(_body, N),
        grid=(grid,),
        in_specs=[
            pl.BlockSpec((_BLK, D), lambda i: (i, 0)),
            full((D, H)), full((1, H)), full((1, H)), full((1, H)),
            full((H, L)), full((1, L)),
            full((Q, L, K)), full((Q, L, K)), full((Q, 3 * L, K)),
            full((L, H)), full((1, H)), full((1, H)), full((1, H)),
            full((H, D)), full((1, D)),
        ],
        out_specs=(
            pl.BlockSpec((_BLK, D), lambda i: (i, 0)),
            pl.BlockSpec((_BLK, Q), lambda i: (i, 0)),
            full((1, 1)), full((1, K)), full((1, 1)),
        ),
        out_shape=out_shapes,
        scratch_shapes=[pltpu.VMEM((Q, K), jnp.float32)],
    )(xf, bf(enc_W1), row(enc_b1), row(enc_g1), row(enc_be1), bf(enc_W2),
      row(enc_b2), cbt, bf(cbt), cbt123,
      bf(dec_W1), row(dec_b1), row(dec_g1), row(dec_be1),
      bf(dec_W2), row(dec_b2))
    return (rec.reshape(B, T, D), idx.reshape(B, T, Q),
            commit[0, 0], perp[0, 0])


# speedup vs baseline: 1.7874x; 1.0861x over previous
import functools

import jax
import jax.numpy as jnp
from jax import lax
from jax.experimental import pallas as pl

_BLK = 512
_NCH = 1
_CH = _BLK // _NCH


def _mm(a, b):
    return jnp.dot(a.astype(jnp.bfloat16), b,
                   preferred_element_type=jnp.float32)


def _ln(x, g, b):
    m = jnp.mean(x, axis=-1, keepdims=True)
    v = jnp.var(x, axis=-1, keepdims=True)
    return (x - m) / jnp.sqrt(v + 1e-5) * g + b


def _split3(cb):
    def trunc(a):
        u = lax.bitcast_convert_type(a, jnp.uint32)
        return lax.bitcast_convert_type(u & jnp.uint32(0xFFFF0000), jnp.float32)
    c1 = trunc(cb)
    r1 = cb - c1
    c2 = trunc(r1)
    r2 = r1 - c2
    return c1.astype(jnp.bfloat16), c2.astype(jnp.bfloat16), r2.astype(jnp.bfloat16)


def _body(n_total, x_ref, ew1, eb1, eg1, ebe1, ew2, eb2, cbt_ref,
          cbtb_ref, cbt123_ref,
          dw1, db1, dg1, dbe1, dw2, db2,
          rec_ref, idx_ref, commit_ref, hist_ref, perp_ref):
    i = pl.program_id(0)
    nsteps = pl.num_programs(0)

    @pl.when(i == 0)
    def _init():
        commit_ref[...] = jnp.zeros_like(commit_ref)
        hist_ref[...] = jnp.zeros_like(hist_ref)
        perp_ref[...] = jnp.zeros_like(perp_ref)

    nq, l_dim, k_dim = cbt_ref.shape
    lane_iota = lax.broadcasted_iota(
        jnp.int32, (_CH, k_dim), 1).astype(jnp.float32)
    dn = (((1,), (1,)), ((), ()))

    zs, res, tq = [], [], []
    for c in range(_NCH):
        x = x_ref[pl.ds(c * _CH, _CH), :]
        h = _ln(_mm(x, ew1[...]) + eb1[...], eg1[...], ebe1[...])
        z = _mm(jax.nn.gelu(h), ew2[...]) + eb2[...]
        zs.append(z)
        res.append(z)
        tq.append(jnp.zeros_like(z))

    commit_sum = jnp.float32(0.0)
    for q in range(nq):
        cbt = cbt_ref[q]
        n_row = jnp.sum(cbt * cbt, axis=0, keepdims=True)
        cbtb = cbtb_ref[q]
        parts_w = cbt123_ref[q]
        for c in range(_NCH):
            r = res[c]
            rsq = jnp.sum(r * r, axis=-1, keepdims=True)
            mm2 = jnp.dot((r + r).astype(jnp.bfloat16), cbtb,
                          preferred_element_type=jnp.float32)
            d = (rsq - mm2) + n_row
            dmin = jnp.min(d, axis=-1, keepdims=True)
            idxf = jnp.min(jnp.where(d == dmin, lane_iota,
                                     jnp.float32(k_dim)), axis=-1)
            idx = idxf.astype(jnp.int32)
            onehot32 = (lane_iota == idxf[:, None]).astype(jnp.float32)
            onehot = onehot32.astype(jnp.bfloat16)
            parts = lax.dot_general(onehot, parts_w, dn,
                                    preferred_element_type=jnp.float32)
            qv = (parts[:, :l_dim] + parts[:, l_dim:2 * l_dim]
                  + parts[:, 2 * l_dim:])
            commit_sum += jnp.sum((qv - r) ** 2)
            tq[c] = tq[c] + qv
            res[c] = r - qv
            idx_ref[pl.ds(c * _CH, _CH), q] = idx
            if q == 0:
                hist_ref[...] += jnp.sum(onehot32, axis=0, keepdims=True)

    for c in range(_NCH):
        quantized = zs[c] + (tq[c] - zs[c])
        hd = _ln(_mm(quantized, dw1[...]) + db1[...], dg1[...], dbe1[...])
        rec_ref[pl.ds(c * _CH, _CH), :] = _mm(jax.nn.gelu(hd), dw2[...]) + db2[...]

    commit_ref[...] += jnp.full((1, 1), 0.25 / (n_total * l_dim)) * commit_sum

    @pl.when(i == nsteps - 1)
    def _finish():
        p = hist_ref[...] / jnp.float32(n_total)
        perp_ref[...] = jnp.exp(-jnp.sum(p * jnp.log(p + 1e-10),
                                         keepdims=True))


def kernel(x, enc_W1, enc_b1, enc_g1, enc_be1, enc_W2, enc_b2, codebooks,
           dec_W1, dec_b1, dec_g1, dec_be1, dec_W2, dec_b2):
    B, T, D = x.shape
    Q, K, L = codebooks.shape
    H = enc_W1.shape[1]
    N = B * T
    xf = x.reshape(N, D)
    cbt = jnp.swapaxes(codebooks, 1, 2)
    cbt1, cbt2, cbt3 = _split3(cbt)
    cbt123 = jnp.concatenate([cbt1, cbt2, cbt3], axis=1)
    bf = lambda a: a.astype(jnp.bfloat16)
    row = lambda a: a.reshape(1, -1)
    grid = N // _BLK

    full = lambda shape: pl.BlockSpec(shape, lambda i: (0,) * len(shape))
    out_shapes = (
        jax.ShapeDtypeStruct((N, D), jnp.float32),
        jax.ShapeDtypeStruct((N, Q), jnp.int32),
        jax.ShapeDtypeStruct((1, 1), jnp.float32),
        jax.ShapeDtypeStruct((1, K), jnp.float32),
        jax.ShapeDtypeStruct((1, 1), jnp.float32),
    )
    rec, idx, commit, _hist, perp = pl.pallas_call(
        functools.partial(_body, N),
        grid=(grid,),
        in_specs=[
            pl.BlockSpec((_BLK, D), lambda i: (i, 0)),
            full((D, H)), full((1, H)), full((1, H)), full((1, H)),
            full((H, L)), full((1, L)),
            full((Q, L, K)), full((Q, L, K)), full((Q, 3 * L, K)),
            full((L, H)), full((1, H)), full((1, H)), full((1, H)),
            full((H, D)), full((1, D)),
        ],
        out_specs=(
            pl.BlockSpec((_BLK, D), lambda i: (i, 0)),
            pl.BlockSpec((_BLK, Q), lambda i: (i, 0)),
            full((1, 1)), full((1, K)), full((1, 1)),
        ),
        out_shape=out_shapes,
    )(xf, bf(enc_W1), row(enc_b1), row(enc_g1), row(enc_be1), bf(enc_W2),
      row(enc_b2), cbt, bf(cbt), cbt123,
      bf(dec_W1), row(dec_b1), row(dec_g1), row(dec_be1),
      bf(dec_W2), row(dec_b2))
    return (rec.reshape(B, T, D), idx.reshape(B, T, Q),
            commit[0, 0], perp[0, 0])


# speedup vs baseline: 1.7876x; 1.0002x over previous
import functools

import jax
import jax.numpy as jnp
from jax import lax
from jax.experimental import pallas as pl

_BLK = 512
_NCH = 1
_CH = _BLK // _NCH


def _mm(a, b):
    return jnp.dot(a.astype(jnp.bfloat16), b,
                   preferred_element_type=jnp.float32)


def _ln(x, g, b):
    m = jnp.mean(x, axis=-1, keepdims=True)
    v = jnp.var(x, axis=-1, keepdims=True)
    return (x - m) / jnp.sqrt(v + 1e-5) * g + b


def _split3(cb):
    def trunc(a):
        u = lax.bitcast_convert_type(a, jnp.uint32)
        return lax.bitcast_convert_type(u & jnp.uint32(0xFFFF0000), jnp.float32)
    c1 = trunc(cb)
    r1 = cb - c1
    c2 = trunc(r1)
    r2 = r1 - c2
    return c1.astype(jnp.bfloat16), c2.astype(jnp.bfloat16), r2.astype(jnp.bfloat16)


def _body(n_total, x_ref, ew1, eb1, eg1, ebe1, ew2, eb2, norms_ref,
          cbtb_ref, cbt123_ref,
          dw1, db1, dg1, dbe1, dw2, db2,
          rec_ref, idx_ref, commit_ref, hist_ref, perp_ref):
    i = pl.program_id(0)
    nsteps = pl.num_programs(0)

    @pl.when(i == 0)
    def _init():
        commit_ref[...] = jnp.zeros_like(commit_ref)
        hist_ref[...] = jnp.zeros_like(hist_ref)
        perp_ref[...] = jnp.zeros_like(perp_ref)

    nq, k_dim = norms_ref.shape
    l_dim = cbtb_ref.shape[1]
    lane_iota = lax.broadcasted_iota(
        jnp.int32, (_CH, k_dim), 1).astype(jnp.float32)
    dn = (((1,), (1,)), ((), ()))

    zs, res, tq = [], [], []
    for c in range(_NCH):
        x = x_ref[pl.ds(c * _CH, _CH), :]
        h = _ln(_mm(x, ew1[...]) + eb1[...], eg1[...], ebe1[...])
        z = _mm(jax.nn.gelu(h), ew2[...]) + eb2[...]
        zs.append(z)
        res.append(z)
        tq.append(jnp.zeros_like(z))

    commit_sum = jnp.float32(0.0)
    for q in range(nq):
        n_row = norms_ref[pl.ds(q, 1), :]
        cbtb = cbtb_ref[q]
        parts_w = cbt123_ref[q]
        for c in range(_NCH):
            r = res[c]
            rsq = jnp.sum(r * r, axis=-1, keepdims=True)
            mm2 = jnp.dot((r + r).astype(jnp.bfloat16), cbtb,
                          preferred_element_type=jnp.float32)
            d = (rsq - mm2) + n_row
            dmin = jnp.min(d, axis=-1, keepdims=True)
            idxf = jnp.min(jnp.where(d == dmin, lane_iota,
                                     jnp.float32(k_dim)), axis=-1)
            idx = idxf.astype(jnp.int32)
            onehot32 = (lane_iota == idxf[:, None]).astype(jnp.float32)
            onehot = onehot32.astype(jnp.bfloat16)
            parts = lax.dot_general(onehot, parts_w, dn,
                                    preferred_element_type=jnp.float32)
            qv = (parts[:, :l_dim] + parts[:, l_dim:2 * l_dim]
                  + parts[:, 2 * l_dim:])
            commit_sum += jnp.sum((qv - r) ** 2)
            tq[c] = tq[c] + qv
            res[c] = r - qv
            idx_ref[pl.ds(c * _CH, _CH), q] = idx
            if q == 0:
                hist_ref[...] += jnp.sum(onehot32, axis=0, keepdims=True)

    for c in range(_NCH):
        quantized = zs[c] + (tq[c] - zs[c])
        hd = _ln(_mm(quantized, dw1[...]) + db1[...], dg1[...], dbe1[...])
        rec_ref[pl.ds(c * _CH, _CH), :] = _mm(jax.nn.gelu(hd), dw2[...]) + db2[...]

    commit_ref[...] += jnp.full((1, 1), 0.25 / (n_total * l_dim)) * commit_sum

    @pl.when(i == nsteps - 1)
    def _finish():
        p = hist_ref[...] / jnp.float32(n_total)
        perp_ref[...] = jnp.exp(-jnp.sum(p * jnp.log(p + 1e-10),
                                         keepdims=True))


def kernel(x, enc_W1, enc_b1, enc_g1, enc_be1, enc_W2, enc_b2, codebooks,
           dec_W1, dec_b1, dec_g1, dec_be1, dec_W2, dec_b2):
    B, T, D = x.shape
    Q, K, L = codebooks.shape
    H = enc_W1.shape[1]
    N = B * T
    xf = x.reshape(N, D)
    cbt = jnp.swapaxes(codebooks, 1, 2)
    cbt1, cbt2, cbt3 = _split3(cbt)
    norms = jnp.sum(codebooks ** 2, axis=-1)
    cbt123 = jnp.concatenate([cbt1, cbt2, cbt3], axis=1)
    bf = lambda a: a.astype(jnp.bfloat16)
    row = lambda a: a.reshape(1, -1)
    grid = N // _BLK

    full = lambda shape: pl.BlockSpec(shape, lambda i: (0,) * len(shape))
    out_shapes = (
        jax.ShapeDtypeStruct((N, D), jnp.float32),
        jax.ShapeDtypeStruct((N, Q), jnp.int32),
        jax.ShapeDtypeStruct((1, 1), jnp.float32),
        jax.ShapeDtypeStruct((1, K), jnp.float32),
        jax.ShapeDtypeStruct((1, 1), jnp.float32),
    )
    rec, idx, commit, _hist, perp = pl.pallas_call(
        functools.partial(_body, N),
        grid=(grid,),
        in_specs=[
            pl.BlockSpec((_BLK, D), lambda i: (i, 0)),
            full((D, H)), full((1, H)), full((1, H)), full((1, H)),
            full((H, L)), full((1, L)),
            full((Q, K)), full((Q, L, K)), full((Q, 3 * L, K)),
            full((L, H)), full((1, H)), full((1, H)), full((1, H)),
            full((H, D)), full((1, D)),
        ],
        out_specs=(
            pl.BlockSpec((_BLK, D), lambda i: (i, 0)),
            pl.BlockSpec((_BLK, Q), lambda i: (i, 0)),
            full((1, 1)), full((1, K)), full((1, 1)),
        ),
        out_shape=out_shapes,
    )(xf, bf(enc_W1), row(enc_b1), row(enc_g1), row(enc_be1), bf(enc_W2),
      row(enc_b2), norms, bf(cbt), cbt123,
      bf(dec_W1), row(dec_b1), row(dec_g1), row(dec_be1),
      bf(dec_W2), row(dec_b2))
    return (rec.reshape(B, T, D), idx.reshape(B, T, Q),
            commit[0, 0], perp[0, 0])
